# baseline (speedup 1.0000x reference)
import functools

import jax
import jax.numpy as jnp
from jax.experimental import pallas as pl
from jax.experimental.pallas import tpu as pltpu

_NUM_SEG = 64
_CH = 64


def _fused_body(nblk, n, ids_ref, x_ref, seg_ref, flat_ref, idc_ref):
    ni = pl.program_id(1)
    nch = nblk // _CH
    c = x_ref.shape[1]

    x = x_ref[0]
    xt = x.T
    flat_ref[0] = xt

    ids = ids_ref[0]
    pos = jax.lax.broadcasted_iota(jnp.int32, ids.shape, 1) + ni * nblk
    valid = pos < n
    ids127_lane = jnp.where(valid, ids, 127)
    hi = jnp.max(jnp.where(valid, ids, -1))
    ids_col = ids127_lane.T
    idc_ref[...] = ids_col

    ids127 = ids_col.reshape(nch, _CH)
    gidlo = jnp.min(ids127, axis=1, keepdims=True)
    gidhi = jnp.max(ids127, axis=1, keepdims=True)

    gmax = jnp.max(xt.reshape(nch, _CH, c), axis=1)

    chunk_iota = jax.lax.broadcasted_iota(jnp.int32, (nch, 1), 0)

    @pl.when(ni == 0)
    def _init():
        seg_ref[...] = jnp.full(seg_ref.shape, -jnp.inf, seg_ref.dtype)

    lo = ids[0, 0]

    def _chunk_range(s):
        hj = jnp.sum((gidhi < s).astype(jnp.int32))
        tj = jnp.sum((gidlo <= s).astype(jnp.int32)) - 1
        return hj, tj

    def body(s, carry):
        hj, tj = carry
        nxt = _chunk_range(s + 1)

        mask_h = idc_ref[pl.ds(hj * _CH, _CH), :] == s
        m_h = jnp.max(jnp.where(mask_h, flat_ref[0, pl.ds(hj * _CH, _CH), :], -jnp.inf),
                      axis=0, keepdims=True)
        mask_t = idc_ref[pl.ds(tj * _CH, _CH), :] == s
        m_t = jnp.max(jnp.where(mask_t, flat_ref[0, pl.ds(tj * _CH, _CH), :], -jnp.inf),
                      axis=0, keepdims=True)

        pure = (chunk_iota > hj) & (chunk_iota < tj)
        m_p = jnp.max(jnp.where(pure, gmax, -jnp.inf), axis=0, keepdims=True)

        m = jnp.maximum(jnp.maximum(m_h, m_t), m_p)
        seg_ref[pl.ds(s, 1), :] = jnp.maximum(seg_ref[pl.ds(s, 1), :], m)
        return nxt

    jax.lax.fori_loop(lo, hi + 1, body, _chunk_range(lo))


def kernel(point_features, cluster_ids, batch_size):
    b, c, n = point_features.shape
    del batch_size

    nblk = 12544
    num_blocks = -(-n // nblk)

    ids3 = cluster_ids.reshape(b, 1, n)

    seg, flat3 = pl.pallas_call(
        functools.partial(_fused_body, nblk, n),
        grid=(b, num_blocks),
        in_specs=[
            pl.BlockSpec((1, 1, nblk), lambda bi, ni: (bi, 0, ni)),
            pl.BlockSpec((1, c, nblk), lambda bi, ni: (bi, 0, ni)),
        ],
        out_specs=[
            pl.BlockSpec((_NUM_SEG, c), lambda bi, ni: (bi, 0)),
            pl.BlockSpec((1, nblk, c), lambda bi, ni: (bi, ni, 0)),
        ],
        out_shape=[
            jax.ShapeDtypeStruct((b * _NUM_SEG, c), point_features.dtype),
            jax.ShapeDtypeStruct((b, n, c), point_features.dtype),
        ],
        scratch_shapes=[pltpu.VMEM((nblk, 1), jnp.int32)],
        compiler_params=pltpu.CompilerParams(
            dimension_semantics=("parallel", "arbitrary"),
        ),
    )(ids3, point_features)
    return seg, flat3.reshape(b * n, c)


# speedup vs baseline: 1.0086x; 1.0086x over previous
import functools

import jax
import jax.numpy as jnp
from jax.experimental import pallas as pl
from jax.experimental.pallas import tpu as pltpu

_NUM_SEG = 64
_CH = 64


def _fused_body(nblk, n, ids_ref, x_ref, seg_ref, flat_ref, idc_ref):
    ni = pl.program_id(1)
    nch = nblk // _CH
    c = x_ref.shape[1]

    x = x_ref[0]
    xt = x.T
    flat_ref[0] = xt

    ids = ids_ref[0]
    pos = jax.lax.broadcasted_iota(jnp.int32, ids.shape, 1) + ni * nblk
    valid = pos < n
    ids127_lane = jnp.where(valid, ids, 127)
    hi = jnp.max(jnp.where(valid, ids, -1))
    ids_col = ids127_lane.T
    idc_ref[...] = ids_col

    ids127 = ids_col.reshape(nch, _CH)
    gidlo = jnp.min(ids127, axis=1, keepdims=True)
    gidhi = jnp.max(ids127, axis=1, keepdims=True)

    gmax = jnp.max(xt.reshape(nch, _CH, c), axis=1)

    chunk_iota = jax.lax.broadcasted_iota(jnp.int32, (nch, 1), 0)

    @pl.when(ni == 0)
    def _init():
        seg_ref[...] = jnp.full(seg_ref.shape, -jnp.inf, seg_ref.dtype)

    lo = ids[0, 0]

    def _chunk_range(s):
        hj = jnp.sum((gidhi < s).astype(jnp.int32))
        tj = jnp.sum((gidlo <= s).astype(jnp.int32)) - 1
        return hj, tj

    def _seg_update(s, hj, tj):
        mask_h = idc_ref[pl.ds(hj * _CH, _CH), :] == s
        m_h = jnp.max(jnp.where(mask_h, flat_ref[0, pl.ds(hj * _CH, _CH), :], -jnp.inf),
                      axis=0, keepdims=True)
        mask_t = idc_ref[pl.ds(tj * _CH, _CH), :] == s
        m_t = jnp.max(jnp.where(mask_t, flat_ref[0, pl.ds(tj * _CH, _CH), :], -jnp.inf),
                      axis=0, keepdims=True)

        pure = (chunk_iota > hj) & (chunk_iota < tj)
        m_p = jnp.max(jnp.where(pure, gmax, -jnp.inf), axis=0, keepdims=True)

        m = jnp.maximum(jnp.maximum(m_h, m_t), m_p)
        seg_ref[pl.ds(s, 1), :] = jnp.maximum(seg_ref[pl.ds(s, 1), :], m)

    def body(k, carry):
        s = lo + 2 * k
        hj0, tj0, hj1, tj1 = carry
        nh0, nt0 = _chunk_range(s + 2)
        nh1, nt1 = _chunk_range(s + 3)

        _seg_update(s, hj0, tj0)

        @pl.when(s + 1 <= hi)
        def _second():
            _seg_update(s + 1, hj1, tj1)

        return (nh0, nt0, nh1, nt1)

    h0, t0 = _chunk_range(lo)
    h1, t1 = _chunk_range(lo + 1)
    jax.lax.fori_loop(0, (hi - lo) // 2 + 1, body, (h0, t0, h1, t1))


def kernel(point_features, cluster_ids, batch_size):
    b, c, n = point_features.shape
    del batch_size

    nblk = 6272
    num_blocks = -(-n // nblk)

    ids3 = cluster_ids.reshape(b, 1, n)

    seg, flat3 = pl.pallas_call(
        functools.partial(_fused_body, nblk, n),
        grid=(b, num_blocks),
        in_specs=[
            pl.BlockSpec((1, 1, nblk), lambda bi, ni: (bi, 0, ni)),
            pl.BlockSpec((1, c, nblk), lambda bi, ni: (bi, 0, ni)),
        ],
        out_specs=[
            pl.BlockSpec((_NUM_SEG, c), lambda bi, ni: (bi, 0)),
            pl.BlockSpec((1, nblk, c), lambda bi, ni: (bi, ni, 0)),
        ],
        out_shape=[
            jax.ShapeDtypeStruct((b * _NUM_SEG, c), point_features.dtype),
            jax.ShapeDtypeStruct((b, n, c), point_features.dtype),
        ],
        scratch_shapes=[pltpu.VMEM((nblk, 1), jnp.int32)],
        compiler_params=pltpu.CompilerParams(
            dimension_semantics=("parallel", "arbitrary"),
        ),
    )(ids3, point_features)
    return seg, flat3.reshape(b * n, c)
